# Initial kernel scaffold; baseline (speedup 1.0000x reference)
import functools

import jax
import jax.numpy as jnp
from jax import lax
from jax.experimental import pallas as pl
from jax.experimental.pallas import tpu as pltpu
from jax.experimental.pallas import tpu_sc as plsc

NC = 2
NS = 16
LANES = 16
NW = NC * NS
CHUNK = 128


def _cdiv(a, b):
  return (a + b - 1) // b


def _zero_rows(rows_v, d):
  def zr(r, _):
    for j in range(d // LANES):
      rows_v[r, pl.ds(j * LANES, LANES)] = jnp.zeros((LANES,), jnp.float32)
    return 0
  lax.fori_loop(0, CHUNK, zr, 0)


def _scale_and_scatter(c, src_v, dst_v, w_v, rows_v, table_hbm, agg_s, sem, d):
  pltpu.async_copy(table_hbm.at[src_v.at[c]], rows_v, sem).wait()

  def scale_e(e, _):
    we = w_v[c, e]
    for j in range(d // LANES):
      sl = pl.ds(j * LANES, LANES)
      rows_v[e, sl] = rows_v[e, sl] * we
    return 0

  lax.fori_loop(0, CHUNK, scale_e, 0)
  pltpu.sync_copy(rows_v, agg_s.at[dst_v.at[c]], add=True)


def _make_sc1(n_nodes, n_pad, nch, d):
  mesh = plsc.VectorSubcoreMesh(
      core_axis_name="c", subcore_axis_name="s", num_cores=NC, num_subcores=NS)
  rows_pt = n_pad // NS
  seg_pt = rows_pt // CHUNK

  @functools.partial(
      pl.kernel,
      out_type=(
          jax.ShapeDtypeStruct((NC, n_pad, d), jnp.float32),
          jax.ShapeDtypeStruct((NC, n_pad), jnp.float32),
          jax.ShapeDtypeStruct((NW, nch, CHUNK), jnp.float32),
      ),
      mesh=mesh,
      scratch_types=[
          pltpu.VMEM((nch, CHUNK), jnp.int32),
          pltpu.VMEM((nch, CHUNK), jnp.int32),
          pltpu.VMEM((nch, CHUNK), jnp.float32),
          pltpu.VMEM((nch, CHUNK), jnp.float32),
          pltpu.VMEM((nch, CHUNK), jnp.float32),
          pltpu.VMEM((n_nodes,), jnp.float32),
          pltpu.VMEM((CHUNK, d), jnp.float32),
          pltpu.VMEM((rows_pt,), jnp.float32),
          pltpu.VMEM_SHARED((n_pad, d), jnp.float32),
          pltpu.VMEM_SHARED((n_pad,), jnp.float32),
          pltpu.SemaphoreType.DMA,
      ],
  )
  def sc1(x_hbm, src_hbm, dst_hbm, et_hbm, ew_hbm, nt_hbm,
          agg_hbm, deg_hbm, w_hbm,
          src_v, dst_v, et_v, ew_v, w_v, nt_v, rows_v, zdeg_v,
          agg_s, deg_s, sem):
    cid = lax.axis_index("c")
    sid = lax.axis_index("s")
    wid = cid * NS + sid

    pltpu.sync_copy(src_hbm.at[wid], src_v)
    pltpu.sync_copy(dst_hbm.at[wid], dst_v)
    pltpu.sync_copy(et_hbm.at[wid], et_v)
    pltpu.sync_copy(ew_hbm.at[wid], ew_v)
    pltpu.sync_copy(nt_hbm, nt_v)

    def chunk_w(c, _):
      def sub(k, _):
        sl = pl.ds(k * LANES, LANES)
        dd = dst_v[c, sl]
        ntg = plsc.load_gather(nt_v, [dd])
        dt = ntg - et_v[c, sl]
        w_v[c, sl] = jnp.exp(-jnp.abs(dt)) * ew_v[c, sl]
        return 0
      lax.fori_loop(0, CHUNK // LANES, sub, 0)
      return 0
    lax.fori_loop(0, nch, chunk_w, 0)

    pltpu.sync_copy(w_v, w_hbm.at[wid])

    _zero_rows(rows_v, d)

    def zd(i, _):
      zdeg_v[pl.ds(i * LANES, LANES)] = jnp.zeros((LANES,), jnp.float32)
      return 0
    lax.fori_loop(0, rows_pt // LANES, zd, 0)

    base = sid * rows_pt
    for s in range(seg_pt):
      pltpu.sync_copy(rows_v, agg_s.at[pl.ds(base + s * CHUNK, CHUNK)])
    pltpu.sync_copy(zdeg_v, deg_s.at[pl.ds(base, rows_pt)])
    plsc.subcore_barrier()

    def deg_chunk(c, _):
      pltpu.sync_copy(w_v.at[c], deg_s.at[dst_v.at[c]], add=True)
      return 0
    lax.fori_loop(0, nch, deg_chunk, 0)

    def agg_chunk(c, _):
      _scale_and_scatter(c, src_v, dst_v, w_v, rows_v, x_hbm, agg_s, sem, d)
      return 0
    lax.fori_loop(0, nch, agg_chunk, 0)

    plsc.subcore_barrier()

    for s in range(seg_pt):
      sl = pl.ds(base + s * CHUNK, CHUNK)
      pltpu.sync_copy(agg_s.at[sl], agg_hbm.at[cid, sl])
    pltpu.sync_copy(deg_s.at[pl.ds(base, rows_pt)],
                    deg_hbm.at[cid, pl.ds(base, rows_pt)])

  return sc1


def _make_sc2(n_pad, nch, d):
  mesh = plsc.VectorSubcoreMesh(
      core_axis_name="c", subcore_axis_name="s", num_cores=NC, num_subcores=NS)
  rows_pt = n_pad // NS
  seg_pt = rows_pt // CHUNK

  @functools.partial(
      pl.kernel,
      out_type=jax.ShapeDtypeStruct((NC, n_pad, d), jnp.float32),
      mesh=mesh,
      scratch_types=[
          pltpu.VMEM((nch, CHUNK), jnp.int32),
          pltpu.VMEM((nch, CHUNK), jnp.int32),
          pltpu.VMEM((nch, CHUNK), jnp.float32),
          pltpu.VMEM((CHUNK, d), jnp.float32),
          pltpu.VMEM_SHARED((n_pad, d), jnp.float32),
          pltpu.SemaphoreType.DMA,
      ],
  )
  def sc2(h_hbm, src_hbm, dst_hbm, w_hbm, agg_hbm,
          src_v, dst_v, w_v, rows_v, agg_s, sem):
    cid = lax.axis_index("c")
    sid = lax.axis_index("s")
    wid = cid * NS + sid

    pltpu.sync_copy(src_hbm.at[wid], src_v)
    pltpu.sync_copy(dst_hbm.at[wid], dst_v)
    pltpu.sync_copy(w_hbm.at[wid], w_v)

    _zero_rows(rows_v, d)
    base = sid * rows_pt
    for s in range(seg_pt):
      pltpu.sync_copy(rows_v, agg_s.at[pl.ds(base + s * CHUNK, CHUNK)])
    plsc.subcore_barrier()

    def agg_chunk(c, _):
      _scale_and_scatter(c, src_v, dst_v, w_v, rows_v, h_hbm, agg_s, sem, d)
      return 0
    lax.fori_loop(0, nch, agg_chunk, 0)

    plsc.subcore_barrier()
    for s in range(seg_pt):
      sl = pl.ds(base + s * CHUNK, CHUNK)
      pltpu.sync_copy(agg_s.at[sl], agg_hbm.at[cid, sl])

  return sc2


def _tc1_body(agg_ref, degt_ref, w_ref, b_ref, out_ref):
  a = agg_ref[0] + agg_ref[1]
  dsum = degt_ref[:, 0:1] + degt_ref[:, 1:2]
  m = a / jnp.maximum(dsum, 1e-6)
  h = jnp.dot(m, w_ref[...], preferred_element_type=jnp.float32) + b_ref[...]
  out_ref[...] = jnp.where(h >= 0, h, 0.01 * h)


def _make_tc1(n_pad, d, blk):
  grid = (n_pad // blk,)
  return pl.pallas_call(
      _tc1_body,
      grid=grid,
      in_specs=[
          pl.BlockSpec((NC, blk, d), lambda i: (0, i, 0)),
          pl.BlockSpec((blk, NC), lambda i: (i, 0)),
          pl.BlockSpec((d, d), lambda i: (0, 0)),
          pl.BlockSpec((1, d), lambda i: (0, 0)),
      ],
      out_specs=pl.BlockSpec((blk, d), lambda i: (i, 0)),
      out_shape=jax.ShapeDtypeStruct((n_pad, d), jnp.float32),
  )


def _make_tc2(n_nodes, n_pad, d):
  def body(agg_ref, degt_ref, w2_ref, b2_ref, gamma_ref, beta_ref,
           wf1_ref, bf1_ref, wf2_ref, bf2_ref, out_ref):
    a = agg_ref[0] + agg_ref[1]
    dsum = degt_ref[:, 0:1] + degt_ref[:, 1:2]
    m = a / jnp.maximum(dsum, 1e-6)
    h = jnp.dot(m, w2_ref[...], preferred_element_type=jnp.float32) + b2_ref[...]
    h = jnp.where(h >= 0, h, 0.01 * h)
    rid = lax.broadcasted_iota(jnp.int32, (n_pad, d), 0)
    msk = rid < n_nodes
    hm = jnp.where(msk, h, 0.0)
    inv_n = 1.0 / n_nodes
    mu = jnp.sum(hm, axis=0, keepdims=True) * inv_n
    ex2 = jnp.sum(hm * hm, axis=0, keepdims=True) * inv_n
    var = ex2 - mu * mu
    hb = (h - mu) / jnp.sqrt(var + 1e-5) * gamma_ref[...] + beta_ref[...]
    hb = jnp.where(hb >= 0, hb, 0.01 * hb)
    h4 = jnp.dot(hb, wf1_ref[...], preferred_element_type=jnp.float32) + bf1_ref[...]
    h4 = jnp.where(h4 >= 0, h4, 0.01 * h4)
    out_ref[...] = (
        jnp.dot(h4, wf2_ref[...], preferred_element_type=jnp.float32)
        + bf2_ref[...])

  return pl.pallas_call(
      body,
      out_shape=jax.ShapeDtypeStruct((n_pad, d), jnp.float32),
  )


def kernel(x, edge_index, edge_time, node_time, edge_weight,
           W1, b1, W2, b2, gamma, beta, Wf1, bf1, Wf2, bf2):
  n_nodes, d = x.shape
  e = edge_index.shape[1]
  out_dim = Wf2.shape[1]

  n_pad = _cdiv(n_nodes, NS * CHUNK) * NS * CHUNK
  nch = _cdiv(e, NW * CHUNK)
  e_pad = nch * NW * CHUNK
  pad = e_pad - e

  src = edge_index[0].astype(jnp.int32)
  dst = edge_index[1].astype(jnp.int32)
  zi = jnp.zeros((pad,), jnp.int32)
  zf = jnp.zeros((pad,), jnp.float32)
  src3 = jnp.concatenate([src, zi]).reshape(NW, nch, CHUNK)
  dst3 = jnp.concatenate([dst, zi]).reshape(NW, nch, CHUNK)
  et3 = jnp.concatenate([edge_time.astype(jnp.float32), zf]).reshape(NW, nch, CHUNK)
  ew3 = jnp.concatenate([edge_weight.astype(jnp.float32), zf]).reshape(NW, nch, CHUNK)

  sc1 = _make_sc1(n_nodes, n_pad, nch, d)
  sc2 = _make_sc2(n_pad, nch, d)
  tc1 = _make_tc1(n_pad, d, 1024)
  tc2 = _make_tc2(n_nodes, n_pad, d)

  agg1, deg, w3 = sc1(x, src3, dst3, et3, ew3, node_time.astype(jnp.float32))
  degt = deg.T

  h1 = tc1(agg1, degt, W1, b1.reshape(1, d))

  agg2 = sc2(h1, src3, dst3, w3)

  wf1p = jnp.pad(Wf1, ((0, 0), (0, d - Wf1.shape[1])))
  bf1p = jnp.pad(bf1, (0, d - bf1.shape[0])).reshape(1, d)
  wf2p = jnp.pad(Wf2, ((0, d - Wf2.shape[0]), (0, d - Wf2.shape[1])))
  bf2p = jnp.pad(bf2, (0, d - bf2.shape[0])).reshape(1, d)

  out = tc2(agg2, degt, W2, b2.reshape(1, d),
            gamma.reshape(1, d), beta.reshape(1, d),
            wf1p, bf1p, wf2p, bf2p)
  return out[:n_nodes, :out_dim]


# speedup vs baseline: 5.8819x; 5.8819x over previous
import functools

import jax
import jax.numpy as jnp
from jax import lax
from jax.experimental import pallas as pl
from jax.experimental.pallas import tpu as pltpu
from jax.experimental.pallas import tpu_sc as plsc

NC = 2
NS = 16
LANES = 16
NW = NC * NS
CHUNK = 128


def _cdiv(a, b):
  return (a + b - 1) // b


def _zero_rows(rows_v, d):
  def zr(r, _):
    for j in range(d // LANES):
      rows_v[r, pl.ds(j * LANES, LANES)] = jnp.zeros((LANES,), jnp.float32)
    return 0
  lax.fori_loop(0, CHUNK, zr, 0)


def _scale_rows(w_c, rows_v, d):
  def scale_g(g, _):
    w16 = w_c[pl.ds(g * LANES, LANES)]
    e0 = g * LANES
    for l in range(LANES):
      we = w16[l]
      for j in range(d // LANES):
        sl = pl.ds(j * LANES, LANES)
        rows_v[e0 + l, sl] = rows_v[e0 + l, sl] * we
    return 0
  lax.fori_loop(0, CHUNK // LANES, scale_g, 0)


def _make_sc1(n_nodes, nt_rows, n_pad, nch, d):
  mesh = plsc.VectorSubcoreMesh(
      core_axis_name="c", subcore_axis_name="s", num_cores=NC, num_subcores=NS)
  rows_pt = n_pad // NS
  seg_pt = rows_pt // CHUNK

  @functools.partial(
      pl.kernel,
      out_type=(
          jax.ShapeDtypeStruct((NC, n_pad, d), jnp.float32),
          jax.ShapeDtypeStruct((NC, n_pad), jnp.float32),
          jax.ShapeDtypeStruct((NW, nch, CHUNK), jnp.float32),
      ),
      mesh=mesh,
      compiler_params=pltpu.CompilerParams(needs_layout_passes=False),
      scratch_types=[
          pltpu.VMEM((CHUNK,), jnp.int32),
          pltpu.VMEM((CHUNK,), jnp.int32),
          pltpu.VMEM((CHUNK,), jnp.float32),
          pltpu.VMEM((CHUNK,), jnp.float32),
          pltpu.VMEM((CHUNK,), jnp.float32),
          pltpu.VMEM((nt_rows, CHUNK), jnp.float32),
          pltpu.VMEM((CHUNK, d), jnp.float32),
          pltpu.VMEM((rows_pt,), jnp.float32),
          pltpu.VMEM_SHARED((n_pad, d), jnp.float32),
          pltpu.VMEM_SHARED((n_pad,), jnp.float32),
          pltpu.SemaphoreType.DMA,
      ],
  )
  def sc1(x_hbm, src_hbm, dst_hbm, et_hbm, ew_hbm, nt_hbm,
          agg_hbm, deg_hbm, w_hbm,
          src_c, dst_c, et_c, ew_c, w_c, nt_v, rows_v, zdeg_v,
          agg_s, deg_s, sem):
    cid = lax.axis_index("c")
    sid = lax.axis_index("s")
    wid = cid * NS + sid

    pltpu.sync_copy(nt_hbm, nt_v)

    _zero_rows(rows_v, d)

    def zd(i, _):
      zdeg_v[pl.ds(i * LANES, LANES)] = jnp.zeros((LANES,), jnp.float32)
      return 0
    lax.fori_loop(0, rows_pt // LANES, zd, 0)

    base = sid * rows_pt
    for s in range(seg_pt):
      pltpu.sync_copy(rows_v, agg_s.at[pl.ds(base + s * CHUNK, CHUNK)])
    pltpu.sync_copy(zdeg_v, deg_s.at[pl.ds(base, rows_pt)])
    plsc.subcore_barrier()

    def chunk_body(c, _):
      pltpu.sync_copy(src_hbm.at[wid, c], src_c)
      pltpu.sync_copy(dst_hbm.at[wid, c], dst_c)
      pltpu.sync_copy(et_hbm.at[wid, c], et_c)
      pltpu.sync_copy(ew_hbm.at[wid, c], ew_c)

      def sub(k, _):
        sl = pl.ds(k * LANES, LANES)
        dd = dst_c[sl]
        ntg = plsc.load_gather(
            nt_v, [lax.shift_right_logical(dd, 7), lax.bitwise_and(dd, 127)])
        dt = ntg - et_c[sl]
        w_c[sl] = jnp.exp(-jnp.abs(dt)) * ew_c[sl]
        return 0
      lax.fori_loop(0, CHUNK // LANES, sub, 0)

      pltpu.sync_copy(w_c, w_hbm.at[wid, c])
      pltpu.sync_copy(w_c, deg_s.at[dst_c], add=True)

      pltpu.async_copy(x_hbm.at[src_c], rows_v, sem).wait()
      _scale_rows(w_c, rows_v, d)
      pltpu.sync_copy(rows_v, agg_s.at[dst_c], add=True)
      return 0
    lax.fori_loop(0, nch, chunk_body, 0)

    plsc.subcore_barrier()

    for s in range(seg_pt):
      sl = pl.ds(base + s * CHUNK, CHUNK)
      pltpu.sync_copy(agg_s.at[sl], agg_hbm.at[cid, sl])
    pltpu.sync_copy(deg_s.at[pl.ds(base, rows_pt)],
                    deg_hbm.at[cid, pl.ds(base, rows_pt)])

  return sc1


def _make_sc2(n_pad, nch, d):
  mesh = plsc.VectorSubcoreMesh(
      core_axis_name="c", subcore_axis_name="s", num_cores=NC, num_subcores=NS)
  rows_pt = n_pad // NS
  seg_pt = rows_pt // CHUNK

  @functools.partial(
      pl.kernel,
      out_type=jax.ShapeDtypeStruct((NC, n_pad, d), jnp.float32),
      mesh=mesh,
      compiler_params=pltpu.CompilerParams(needs_layout_passes=False),
      scratch_types=[
          pltpu.VMEM((CHUNK,), jnp.int32),
          pltpu.VMEM((CHUNK,), jnp.int32),
          pltpu.VMEM((CHUNK,), jnp.float32),
          pltpu.VMEM((CHUNK, d), jnp.float32),
          pltpu.VMEM_SHARED((n_pad, d), jnp.float32),
          pltpu.SemaphoreType.DMA,
      ],
  )
  def sc2(h_hbm, src_hbm, dst_hbm, w_hbm, agg_hbm,
          src_c, dst_c, w_c, rows_v, agg_s, sem):
    cid = lax.axis_index("c")
    sid = lax.axis_index("s")
    wid = cid * NS + sid

    _zero_rows(rows_v, d)
    base = sid * rows_pt
    for s in range(seg_pt):
      pltpu.sync_copy(rows_v, agg_s.at[pl.ds(base + s * CHUNK, CHUNK)])
    plsc.subcore_barrier()

    def chunk_body(c, _):
      pltpu.sync_copy(src_hbm.at[wid, c], src_c)
      pltpu.sync_copy(dst_hbm.at[wid, c], dst_c)
      pltpu.sync_copy(w_hbm.at[wid, c], w_c)
      pltpu.async_copy(h_hbm.at[src_c], rows_v, sem).wait()
      _scale_rows(w_c, rows_v, d)
      pltpu.sync_copy(rows_v, agg_s.at[dst_c], add=True)
      return 0
    lax.fori_loop(0, nch, chunk_body, 0)

    plsc.subcore_barrier()
    for s in range(seg_pt):
      sl = pl.ds(base + s * CHUNK, CHUNK)
      pltpu.sync_copy(agg_s.at[sl], agg_hbm.at[cid, sl])

  return sc2


def _tc1_body(agg_ref, degt_ref, w_ref, b_ref, out_ref):
  a = agg_ref[0] + agg_ref[1]
  dsum = degt_ref[:, 0:1] + degt_ref[:, 1:2]
  m = a / jnp.maximum(dsum, 1e-6)
  h = jnp.dot(m, w_ref[...], preferred_element_type=jnp.float32) + b_ref[...]
  out_ref[...] = jnp.where(h >= 0, h, 0.01 * h)


def _make_tc1(n_pad, d, blk):
  grid = (n_pad // blk,)
  return pl.pallas_call(
      _tc1_body,
      grid=grid,
      in_specs=[
          pl.BlockSpec((NC, blk, d), lambda i: (0, i, 0)),
          pl.BlockSpec((blk, NC), lambda i: (i, 0)),
          pl.BlockSpec((d, d), lambda i: (0, 0)),
          pl.BlockSpec((1, d), lambda i: (0, 0)),
      ],
      out_specs=pl.BlockSpec((blk, d), lambda i: (i, 0)),
      out_shape=jax.ShapeDtypeStruct((n_pad, d), jnp.float32),
  )


def _make_tc2(n_nodes, n_pad, d):
  def body(agg_ref, degt_ref, w2_ref, b2_ref, gamma_ref, beta_ref,
           wf1_ref, bf1_ref, wf2_ref, bf2_ref, out_ref):
    a = agg_ref[0] + agg_ref[1]
    dsum = degt_ref[:, 0:1] + degt_ref[:, 1:2]
    m = a / jnp.maximum(dsum, 1e-6)
    h = jnp.dot(m, w2_ref[...], preferred_element_type=jnp.float32) + b2_ref[...]
    h = jnp.where(h >= 0, h, 0.01 * h)
    rid = lax.broadcasted_iota(jnp.int32, (n_pad, d), 0)
    msk = rid < n_nodes
    hm = jnp.where(msk, h, 0.0)
    inv_n = 1.0 / n_nodes
    mu = jnp.sum(hm, axis=0, keepdims=True) * inv_n
    ex2 = jnp.sum(hm * hm, axis=0, keepdims=True) * inv_n
    var = ex2 - mu * mu
    hb = (h - mu) / jnp.sqrt(var + 1e-5) * gamma_ref[...] + beta_ref[...]
    hb = jnp.where(hb >= 0, hb, 0.01 * hb)
    h4 = jnp.dot(hb, wf1_ref[...], preferred_element_type=jnp.float32) + bf1_ref[...]
    h4 = jnp.where(h4 >= 0, h4, 0.01 * h4)
    out_ref[...] = (
        jnp.dot(h4, wf2_ref[...], preferred_element_type=jnp.float32)
        + bf2_ref[...])

  return pl.pallas_call(
      body,
      out_shape=jax.ShapeDtypeStruct((n_pad, d), jnp.float32),
  )


def kernel(x, edge_index, edge_time, node_time, edge_weight,
           W1, b1, W2, b2, gamma, beta, Wf1, bf1, Wf2, bf2):
  n_nodes, d = x.shape
  e = edge_index.shape[1]
  out_dim = Wf2.shape[1]

  n_pad = _cdiv(n_nodes, NS * CHUNK) * NS * CHUNK
  nch = _cdiv(e, NW * CHUNK)
  e_pad = nch * NW * CHUNK
  pad = e_pad - e

  src = edge_index[0].astype(jnp.int32)
  dst = edge_index[1].astype(jnp.int32)
  zi = jnp.zeros((pad,), jnp.int32)
  zf = jnp.zeros((pad,), jnp.float32)
  src3 = jnp.concatenate([src, zi]).reshape(NW, nch, CHUNK)
  dst3 = jnp.concatenate([dst, zi]).reshape(NW, nch, CHUNK)
  et3 = jnp.concatenate([edge_time.astype(jnp.float32), zf]).reshape(NW, nch, CHUNK)
  ew3 = jnp.concatenate([edge_weight.astype(jnp.float32), zf]).reshape(NW, nch, CHUNK)

  nt_rows = _cdiv(n_nodes, CHUNK)
  nt2 = jnp.pad(node_time.astype(jnp.float32),
                (0, nt_rows * CHUNK - n_nodes)).reshape(nt_rows, CHUNK)
  sc1 = _make_sc1(n_nodes, nt_rows, n_pad, nch, d)
  sc2 = _make_sc2(n_pad, nch, d)
  tc1 = _make_tc1(n_pad, d, 1024)
  tc2 = _make_tc2(n_nodes, n_pad, d)

  agg1, deg, w3 = sc1(x, src3, dst3, et3, ew3, nt2)
  degt = deg.T

  h1 = tc1(agg1, degt, W1, b1.reshape(1, d))

  agg2 = sc2(h1, src3, dst3, w3)

  wf1p = jnp.pad(Wf1, ((0, 0), (0, d - Wf1.shape[1])))
  bf1p = jnp.pad(bf1, (0, d - bf1.shape[0])).reshape(1, d)
  wf2p = jnp.pad(Wf2, ((0, d - Wf2.shape[0]), (0, d - Wf2.shape[1])))
  bf2p = jnp.pad(bf2, (0, d - bf2.shape[0])).reshape(1, d)

  out = tc2(agg2, degt, W2, b2.reshape(1, d),
            gamma.reshape(1, d), beta.reshape(1, d),
            wf1p, bf1p, wf2p, bf2p)
  return out[:n_nodes, :out_dim]


# speedup vs baseline: 6.0958x; 1.0364x over previous
import functools

import jax
import jax.numpy as jnp
from jax import lax
from jax.experimental import pallas as pl
from jax.experimental.pallas import tpu as pltpu
from jax.experimental.pallas import tpu_sc as plsc

NC = 2
NS = 16
LANES = 16
NW = NC * NS
CHUNK = 128


def _cdiv(a, b):
  return (a + b - 1) // b


def _zero_rows(rows_v, d):
  def zr(r, _):
    for j in range(d // LANES):
      rows_v[r, pl.ds(j * LANES, LANES)] = jnp.zeros((LANES,), jnp.float32)
    return 0
  lax.fori_loop(0, CHUNK, zr, 0)


def _scale_rows(rows_b, load_w16, d):
  def scale_g(g, _):
    w16 = load_w16(g)
    e0 = g * LANES
    for l in range(LANES):
      we = w16[l]
      for j in range(d // LANES):
        sl = pl.ds(j * LANES, LANES)
        rows_b[e0 + l, sl] = rows_b[e0 + l, sl] * we
    return 0
  lax.fori_loop(0, CHUNK // LANES, scale_g, 0)


def _make_sc1(n_nodes, nt_rows, n_pad, nch, d):
  mesh = plsc.VectorSubcoreMesh(
      core_axis_name="c", subcore_axis_name="s", num_cores=NC, num_subcores=NS)
  rows_pt = n_pad // NS
  seg_pt = rows_pt // CHUNK
  assert nch % 2 == 0

  @functools.partial(
      pl.kernel,
      out_type=(
          jax.ShapeDtypeStruct((NC, n_pad, d), jnp.float32),
          jax.ShapeDtypeStruct((NC, n_pad), jnp.float32),
          jax.ShapeDtypeStruct((NW, nch, CHUNK), jnp.float32),
      ),
      mesh=mesh,
      compiler_params=pltpu.CompilerParams(needs_layout_passes=False),
      scratch_types=[
          pltpu.VMEM((4, CHUNK), jnp.int32),
          pltpu.VMEM((4, CHUNK), jnp.int32),
          pltpu.VMEM((CHUNK,), jnp.float32),
          pltpu.VMEM((CHUNK,), jnp.float32),
          pltpu.VMEM((CHUNK, d), jnp.float32),
          pltpu.VMEM((CHUNK, d), jnp.float32),
          pltpu.VMEM((nt_rows, CHUNK), jnp.float32),
          pltpu.VMEM((rows_pt,), jnp.float32),
          pltpu.VMEM_SHARED((n_pad, d), jnp.float32),
          pltpu.VMEM_SHARED((n_pad,), jnp.float32),
          pltpu.SemaphoreType.DMA,
          pltpu.SemaphoreType.DMA,
          pltpu.SemaphoreType.DMA,
          pltpu.SemaphoreType.DMA,
      ],
  )
  def sc1(x_hbm, edata_hbm, nt_hbm,
          agg_hbm, deg_hbm, w_hbm,
          eb0, eb1, w0, w1, rows0, rows1, nt_v, zdeg_v,
          agg_s, deg_s, gsem0, gsem1, ssem0, ssem1):
    cid = lax.axis_index("c")
    sid = lax.axis_index("s")
    wid = cid * NS + sid
    eb = (eb0, eb1)
    wc = (w0, w1)
    rows = (rows0, rows1)
    gsem = (gsem0, gsem1)
    ssem = (ssem0, ssem1)

    pltpu.sync_copy(nt_hbm, nt_v)

    _zero_rows(rows0, d)

    def zd(i, _):
      zdeg_v[pl.ds(i * LANES, LANES)] = jnp.zeros((LANES,), jnp.float32)
      return 0
    lax.fori_loop(0, rows_pt // LANES, zd, 0)

    base = sid * rows_pt
    for s in range(seg_pt):
      pltpu.sync_copy(rows0, agg_s.at[pl.ds(base + s * CHUNK, CHUNK)])
    pltpu.sync_copy(zdeg_v, deg_s.at[pl.ds(base, rows_pt)])
    plsc.subcore_barrier()

    def load_edata(k, b):
      pltpu.sync_copy(edata_hbm.at[wid, k], eb[b])

    def start_gather(b):
      pltpu.async_copy(x_hbm.at[eb[b].at[0]], rows[b], gsem[b])

    def wait_scatter(b):
      pltpu.make_async_copy(rows[b], agg_s.at[eb[b].at[1]], ssem[b]).wait()

    def process(k, b):
      pltpu.make_async_copy(x_hbm.at[eb[b].at[0]], rows[b], gsem[b]).wait()

      def sub(kk, _):
        sl = pl.ds(kk * LANES, LANES)
        dd = eb[b][1, sl]
        ntg = plsc.load_gather(
            nt_v, [lax.shift_right_logical(dd, 7), lax.bitwise_and(dd, 127)])
        dt = ntg - plsc.bitcast(eb[b][2, sl], jnp.float32)
        wcb = wc[b]
        wcb[sl] = jnp.exp(-jnp.abs(dt)) * plsc.bitcast(eb[b][3, sl], jnp.float32)
        return 0
      lax.fori_loop(0, CHUNK // LANES, sub, 0)

      pltpu.sync_copy(wc[b], w_hbm.at[wid, k])
      pltpu.sync_copy(wc[b], deg_s.at[eb[b].at[1]], add=True)

      def w16_of(g):
        return wc[b][pl.ds(g * LANES, LANES)]
      _scale_rows(rows[b], w16_of, d)
      pltpu.async_copy(rows[b], agg_s.at[eb[b].at[1]], ssem[b], add=True)

    load_edata(0, 0)
    start_gather(0)
    load_edata(1, 1)
    start_gather(1)
    process(0, 0)

    def pair(t, _):
      k = 2 * t + 1
      wait_scatter(0)
      load_edata(k + 1, 0)
      start_gather(0)
      process(k, 1)
      wait_scatter(1)
      load_edata(k + 2, 1)
      start_gather(1)
      process(k + 1, 0)
      return 0
    lax.fori_loop(0, nch // 2 - 1, pair, 0)

    wait_scatter(0)
    process(nch - 1, 1)
    wait_scatter(1)

    plsc.subcore_barrier()

    for s in range(seg_pt):
      sl = pl.ds(base + s * CHUNK, CHUNK)
      pltpu.sync_copy(agg_s.at[sl], agg_hbm.at[cid, sl])
    pltpu.sync_copy(deg_s.at[pl.ds(base, rows_pt)],
                    deg_hbm.at[cid, pl.ds(base, rows_pt)])

  return sc1


def _make_sc2(n_pad, nch, d):
  mesh = plsc.VectorSubcoreMesh(
      core_axis_name="c", subcore_axis_name="s", num_cores=NC, num_subcores=NS)
  rows_pt = n_pad // NS
  seg_pt = rows_pt // CHUNK
  assert nch % 2 == 0

  @functools.partial(
      pl.kernel,
      out_type=jax.ShapeDtypeStruct((NC, n_pad, d), jnp.float32),
      mesh=mesh,
      compiler_params=pltpu.CompilerParams(needs_layout_passes=False),
      scratch_types=[
          pltpu.VMEM((3, CHUNK), jnp.int32),
          pltpu.VMEM((3, CHUNK), jnp.int32),
          pltpu.VMEM((CHUNK, d), jnp.float32),
          pltpu.VMEM((CHUNK, d), jnp.float32),
          pltpu.VMEM_SHARED((n_pad, d), jnp.float32),
          pltpu.SemaphoreType.DMA,
          pltpu.SemaphoreType.DMA,
          pltpu.SemaphoreType.DMA,
          pltpu.SemaphoreType.DMA,
      ],
  )
  def sc2(h_hbm, edata_hbm, agg_hbm,
          eb0, eb1, rows0, rows1, agg_s, gsem0, gsem1, ssem0, ssem1):
    cid = lax.axis_index("c")
    sid = lax.axis_index("s")
    wid = cid * NS + sid
    eb = (eb0, eb1)
    rows = (rows0, rows1)
    gsem = (gsem0, gsem1)
    ssem = (ssem0, ssem1)

    _zero_rows(rows0, d)
    base = sid * rows_pt
    for s in range(seg_pt):
      pltpu.sync_copy(rows0, agg_s.at[pl.ds(base + s * CHUNK, CHUNK)])
    plsc.subcore_barrier()

    def load_edata(k, b):
      pltpu.sync_copy(edata_hbm.at[wid, k], eb[b])

    def start_gather(b):
      pltpu.async_copy(h_hbm.at[eb[b].at[0]], rows[b], gsem[b])

    def wait_scatter(b):
      pltpu.make_async_copy(rows[b], agg_s.at[eb[b].at[1]], ssem[b]).wait()

    def process(b):
      pltpu.make_async_copy(h_hbm.at[eb[b].at[0]], rows[b], gsem[b]).wait()

      def w16_of(g):
        return plsc.bitcast(eb[b][2, pl.ds(g * LANES, LANES)], jnp.float32)
      _scale_rows(rows[b], w16_of, d)
      pltpu.async_copy(rows[b], agg_s.at[eb[b].at[1]], ssem[b], add=True)

    load_edata(0, 0)
    start_gather(0)
    load_edata(1, 1)
    start_gather(1)
    process(0)

    def pair(t, _):
      k = 2 * t + 1
      wait_scatter(0)
      load_edata(k + 1, 0)
      start_gather(0)
      process(1)
      wait_scatter(1)
      load_edata(k + 2, 1)
      start_gather(1)
      process(0)
      return 0
    lax.fori_loop(0, nch // 2 - 1, pair, 0)

    wait_scatter(0)
    process(1)
    wait_scatter(1)

    plsc.subcore_barrier()
    for s in range(seg_pt):
      sl = pl.ds(base + s * CHUNK, CHUNK)
      pltpu.sync_copy(agg_s.at[sl], agg_hbm.at[cid, sl])

  return sc2


def _tc1_body(agg_ref, degt_ref, w_ref, b_ref, out_ref):
  a = agg_ref[0] + agg_ref[1]
  dsum = degt_ref[:, 0:1] + degt_ref[:, 1:2]
  m = a / jnp.maximum(dsum, 1e-6)
  h = jnp.dot(m, w_ref[...], preferred_element_type=jnp.float32) + b_ref[...]
  out_ref[...] = jnp.where(h >= 0, h, 0.01 * h)


def _make_tc1(n_pad, d, blk):
  grid = (n_pad // blk,)
  return pl.pallas_call(
      _tc1_body,
      grid=grid,
      in_specs=[
          pl.BlockSpec((NC, blk, d), lambda i: (0, i, 0)),
          pl.BlockSpec((blk, NC), lambda i: (i, 0)),
          pl.BlockSpec((d, d), lambda i: (0, 0)),
          pl.BlockSpec((1, d), lambda i: (0, 0)),
      ],
      out_specs=pl.BlockSpec((blk, d), lambda i: (i, 0)),
      out_shape=jax.ShapeDtypeStruct((n_pad, d), jnp.float32),
  )


def _make_tc2(n_nodes, n_pad, d):
  def body(agg_ref, degt_ref, w2_ref, b2_ref, gamma_ref, beta_ref,
           wf1_ref, bf1_ref, wf2_ref, bf2_ref, out_ref):
    a = agg_ref[0] + agg_ref[1]
    dsum = degt_ref[:, 0:1] + degt_ref[:, 1:2]
    m = a / jnp.maximum(dsum, 1e-6)
    h = jnp.dot(m, w2_ref[...], preferred_element_type=jnp.float32) + b2_ref[...]
    h = jnp.where(h >= 0, h, 0.01 * h)
    rid = lax.broadcasted_iota(jnp.int32, (n_pad, d), 0)
    msk = rid < n_nodes
    hm = jnp.where(msk, h, 0.0)
    inv_n = 1.0 / n_nodes
    mu = jnp.sum(hm, axis=0, keepdims=True) * inv_n
    ex2 = jnp.sum(hm * hm, axis=0, keepdims=True) * inv_n
    var = ex2 - mu * mu
    hb = (h - mu) / jnp.sqrt(var + 1e-5) * gamma_ref[...] + beta_ref[...]
    hb = jnp.where(hb >= 0, hb, 0.01 * hb)
    h4 = jnp.dot(hb, wf1_ref[...], preferred_element_type=jnp.float32) + bf1_ref[...]
    h4 = jnp.where(h4 >= 0, h4, 0.01 * h4)
    out_ref[...] = (
        jnp.dot(h4, wf2_ref[...], preferred_element_type=jnp.float32)
        + bf2_ref[...])

  return pl.pallas_call(
      body,
      out_shape=jax.ShapeDtypeStruct((n_pad, d), jnp.float32),
  )


def kernel(x, edge_index, edge_time, node_time, edge_weight,
           W1, b1, W2, b2, gamma, beta, Wf1, bf1, Wf2, bf2):
  n_nodes, d = x.shape
  e = edge_index.shape[1]
  out_dim = Wf2.shape[1]

  n_pad = _cdiv(n_nodes, NS * CHUNK) * NS * CHUNK
  nch = 2 * _cdiv(e, NW * CHUNK * 2)
  e_pad = nch * NW * CHUNK
  pad = e_pad - e

  src = edge_index[0].astype(jnp.int32)
  dst = edge_index[1].astype(jnp.int32)
  zi = jnp.zeros((pad,), jnp.int32)
  zf = jnp.zeros((pad,), jnp.float32)
  src3 = jnp.concatenate([src, zi]).reshape(NW, nch, CHUNK)
  dst3 = jnp.concatenate([dst, zi]).reshape(NW, nch, CHUNK)
  et3 = jnp.concatenate([edge_time.astype(jnp.float32), zf]).reshape(NW, nch, CHUNK)
  ew3 = jnp.concatenate([edge_weight.astype(jnp.float32), zf]).reshape(NW, nch, CHUNK)
  edata1 = jnp.stack([
      src3, dst3,
      lax.bitcast_convert_type(et3, jnp.int32),
      lax.bitcast_convert_type(ew3, jnp.int32),
  ], axis=2)

  nt_rows = _cdiv(n_nodes, CHUNK)
  nt2 = jnp.pad(node_time.astype(jnp.float32),
                (0, nt_rows * CHUNK - n_nodes)).reshape(nt_rows, CHUNK)
  sc1 = _make_sc1(n_nodes, nt_rows, n_pad, nch, d)
  sc2 = _make_sc2(n_pad, nch, d)
  tc1 = _make_tc1(n_pad, d, 1024)
  tc2 = _make_tc2(n_nodes, n_pad, d)

  agg1, deg, w3 = sc1(x, edata1, nt2)
  degt = deg.T

  h1 = tc1(agg1, degt, W1, b1.reshape(1, d))

  edata2 = jnp.stack(
      [src3, dst3, lax.bitcast_convert_type(w3, jnp.int32)], axis=2)
  agg2 = sc2(h1, edata2)

  wf1p = jnp.pad(Wf1, ((0, 0), (0, d - Wf1.shape[1])))
  bf1p = jnp.pad(bf1, (0, d - bf1.shape[0])).reshape(1, d)
  wf2p = jnp.pad(Wf2, ((0, d - Wf2.shape[0]), (0, d - Wf2.shape[1])))
  bf2p = jnp.pad(bf2, (0, d - bf2.shape[0])).reshape(1, d)

  out = tc2(agg2, degt, W2, b2.reshape(1, d),
            gamma.reshape(1, d), beta.reshape(1, d),
            wf1p, bf1p, wf2p, bf2p)
  return out[:n_nodes, :out_dim]
